# baseline (speedup 1.0000x reference)
import functools

import jax
import jax.numpy as jnp
from jax import lax
from jax.experimental import pallas as pl
from jax.experimental.pallas import tpu as pltpu
from jax.experimental.pallas import tpu_sc as plsc

D = 128
B = 16384
NC = 2
NS = 16
NW = NC * NS
BW = B // NW
NCHUNK = 4
CW = BW // NCHUNK

_mesh = plsc.VectorSubcoreMesh(core_axis_name="c", subcore_axis_name="s")


@functools.partial(
    pl.kernel,
    out_type=[
        jax.ShapeDtypeStruct((B, D), jnp.float32),
        jax.ShapeDtypeStruct((B,), jnp.int32),
        jax.ShapeDtypeStruct((16,), jnp.float32),
    ],
    mesh=_mesh,
    scratch_types=[
        pltpu.VMEM((BW + 16,), jnp.int32),
        pltpu.VMEM((BW + 16,), jnp.int32),
        [pltpu.VMEM((CW, D), jnp.float32) for _ in range(NCHUNK)],
        pltpu.VMEM((16,), jnp.float32),
        [pltpu.SemaphoreType.DMA for _ in range(NCHUNK)],
        pltpu.SemaphoreType.DMA,
        pltpu.SemaphoreType.DMA,
    ],
)
def _gather_kernel(n_id_hbm, lu_hbm, init_hbm, mem_hbm,
                   out_mem, out_lu, out_loss,
                   idx_v, lu_v, rows, loss_v, gsems, wsem, lsem):
    wid = lax.axis_index("s") * NC + lax.axis_index("c")
    base = wid * BW

    idx_w = idx_v.at[pl.ds(0, BW)]
    lu_w = lu_v.at[pl.ds(0, BW)]
    pltpu.sync_copy(n_id_hbm.at[pl.ds(base, BW)], idx_w)
    c_lu = pltpu.async_copy(lu_hbm.at[idx_w], lu_w, lsem)
    c_rows = [
        pltpu.async_copy(
            init_hbm.at[idx_v.at[pl.ds(c * CW, CW)]], rows[c], gsems[c])
        for c in range(NCHUNK)
    ]

    c_w = []
    for c in range(NCHUNK):
        c_rows[c].wait()
        c_w.append(pltpu.async_copy(
            rows[c], out_mem.at[pl.ds(base + c * CW, CW)], wsem))

    c_lu.wait()
    pltpu.sync_copy(lu_w, out_lu.at[pl.ds(base, BW)])

    def _or_stale(i, acc):
        chunk = lu_v[pl.ds(i * 16, 16)]
        return acc | (chunk ^ jnp.full((16,), -1, jnp.int32))

    acc = lax.fori_loop(0, BW // 16, _or_stale, jnp.zeros((16,), jnp.int32))
    n_stale = acc[0]
    for j in range(1, 16):
        n_stale = n_stale | acc[j]

    for c in range(NCHUNK):
        c_w[c].wait()

    @pl.when(n_stale != 0)
    def _general_path():
        for c in range(NCHUNK):
            def _fix_row_c(r, carry, c=c):
                g = c * CW + r
                lur = lu_v[pl.ds(g, 16)][0]
                nid_r = idx_v[pl.ds(g, 16)][0]

                @pl.when(lur != -1)
                def _():
                    def _copy_mem_row(sem):
                        pltpu.async_copy(
                            mem_hbm.at[nid_r], rows[c].at[r], sem).wait()
                    pl.run_scoped(_copy_mem_row, pltpu.SemaphoreType.DMA)
                return carry

            lax.fori_loop(0, CW, _fix_row_c, jnp.int32(0))
            pltpu.sync_copy(rows[c], out_mem.at[pl.ds(base + c * CW, CW)])

    @pl.when(wid == 0)
    def _write_loss():
        loss_v[...] = jnp.zeros((16,), jnp.float32)
        pltpu.sync_copy(loss_v, out_loss)


def kernel(n_id, memory, last_update, init_memory, W_ih, W_hh, b_ih, b_hh):
    mem, lu, loss_v = _gather_kernel(n_id, last_update, init_memory, memory)
    return mem, lu, loss_v[0]


# speedup vs baseline: 1.0021x; 1.0021x over previous
import functools

import jax
import jax.numpy as jnp
from jax import lax
from jax.experimental import pallas as pl
from jax.experimental.pallas import tpu as pltpu
from jax.experimental.pallas import tpu_sc as plsc

D = 128
B = 16384
NC = 2
NS = 16
NW = NC * NS
BW = B // NW
NCHUNK = 4
CW = BW // NCHUNK

_mesh = plsc.VectorSubcoreMesh(core_axis_name="c", subcore_axis_name="s")


@functools.partial(
    pl.kernel,
    out_type=[
        jax.ShapeDtypeStruct((B, D), jnp.float32),
        jax.ShapeDtypeStruct((B,), jnp.int32),
        jax.ShapeDtypeStruct((16,), jnp.float32),
    ],
    mesh=_mesh,
    scratch_types=[
        pltpu.VMEM((BW + 16,), jnp.int32),
        pltpu.VMEM((BW + 16,), jnp.int32),
        [pltpu.VMEM((CW, D), jnp.float32) for _ in range(NCHUNK)],
        pltpu.VMEM((16,), jnp.float32),
        [pltpu.SemaphoreType.DMA for _ in range(NCHUNK)],
        pltpu.SemaphoreType.DMA,
        pltpu.SemaphoreType.DMA,
    ],
)
def _gather_kernel(n_id_hbm, lu_hbm, init_hbm, mem_hbm,
                   out_mem, out_lu, out_loss,
                   idx_v, lu_v, rows, loss_v, gsems, wsem, lsem):
    wid = lax.axis_index("s") * NC + lax.axis_index("c")
    base = wid * BW

    idx_w = idx_v.at[pl.ds(0, BW)]
    lu_w = lu_v.at[pl.ds(0, BW)]
    pltpu.sync_copy(n_id_hbm.at[pl.ds(base, BW)], idx_w)
    c_rows = [
        pltpu.async_copy(
            init_hbm.at[idx_v.at[pl.ds(c * CW, CW)]], rows[c], gsems[c])
        for c in range(NCHUNK)
    ]
    c_lu = pltpu.async_copy(lu_hbm.at[idx_w], lu_w, lsem)

    c_w = []
    for c in range(NCHUNK):
        c_rows[c].wait()
        c_w.append(pltpu.async_copy(
            rows[c], out_mem.at[pl.ds(base + c * CW, CW)], wsem))

    c_lu.wait()
    pltpu.sync_copy(lu_w, out_lu.at[pl.ds(base, BW)])

    def _or_stale(i, acc):
        chunk = lu_v[pl.ds(i * 16, 16)]
        return acc | (chunk ^ jnp.full((16,), -1, jnp.int32))

    acc = lax.fori_loop(0, BW // 16, _or_stale, jnp.zeros((16,), jnp.int32))
    n_stale = acc[0]
    for j in range(1, 16):
        n_stale = n_stale | acc[j]

    for c in range(NCHUNK):
        c_w[c].wait()

    @pl.when(n_stale != 0)
    def _general_path():
        for c in range(NCHUNK):
            def _fix_row_c(r, carry, c=c):
                g = c * CW + r
                lur = lu_v[pl.ds(g, 16)][0]
                nid_r = idx_v[pl.ds(g, 16)][0]

                @pl.when(lur != -1)
                def _():
                    def _copy_mem_row(sem):
                        pltpu.async_copy(
                            mem_hbm.at[nid_r], rows[c].at[r], sem).wait()
                    pl.run_scoped(_copy_mem_row, pltpu.SemaphoreType.DMA)
                return carry

            lax.fori_loop(0, CW, _fix_row_c, jnp.int32(0))
            pltpu.sync_copy(rows[c], out_mem.at[pl.ds(base + c * CW, CW)])

    @pl.when(wid == 0)
    def _write_loss():
        loss_v[...] = jnp.zeros((16,), jnp.float32)
        pltpu.sync_copy(loss_v, out_loss)


def kernel(n_id, memory, last_update, init_memory, W_ih, W_hh, b_ih, b_hh):
    mem, lu, loss_v = _gather_kernel(n_id, last_update, init_memory, memory)
    return mem, lu, loss_v[0]
